# baseline (speedup 1.0000x reference)
import functools

import jax
import jax.numpy as jnp
from jax import lax
from jax.experimental import pallas as pl
from jax.experimental.pallas import tpu as pltpu
from jax.experimental.pallas import tpu_sc as plsc

_L = 16


def _gate_body(x_ref, w1t_ref, b1_ref, w2t_ref, b2_ref, y_ref, *, scale):
    y = jnp.sum(x_ref[...], axis=0) * scale
    h = jnp.dot(y, w1t_ref[...], preferred_element_type=jnp.float32)
    h = h + b1_ref[...]
    h = jnp.where(h >= 0, h, 0.01 * h)
    h = jnp.dot(h, w2t_ref[...], preferred_element_type=jnp.float32)
    h = h + b2_ref[...]
    y_ref[...] = jax.nn.sigmoid(h)


def _gate_tc(xt, W1, b1, W2, b2, b0, bn):
    HW, B, C = xt.shape
    HID = W1.shape[0]
    BB = 16
    grid = bn // BB
    blk0 = b0 // BB
    return pl.pallas_call(
        functools.partial(_gate_body, scale=1.0 / HW),
        grid=(grid,),
        in_specs=[
            pl.BlockSpec((HW, BB, C), lambda i: (0, blk0 + i, 0)),
            pl.BlockSpec((C, HID), lambda i: (0, 0)),
            pl.BlockSpec((1, HID), lambda i: (0, 0)),
            pl.BlockSpec((HID, C), lambda i: (0, 0)),
            pl.BlockSpec((1, C), lambda i: (0, 0)),
        ],
        out_specs=pl.BlockSpec((BB, C), lambda i: (i, 0)),
        out_shape=jax.ShapeDtypeStruct((bn, C), jnp.float32),
    )(xt, W1.T, b1.reshape(1, HID), W2.T, b2.reshape(1, C))


def _make_sc_select(B, C, K):
    NW = 32
    SPW = B // NW
    NCH = C // _L

    mesh = plsc.VectorSubcoreMesh(
        core_axis_name="c", subcore_axis_name="s", num_cores=2, num_subcores=16
    )

    def body(yi_hbm, sel_hbm, exc_hbm, yv, selv, excv, hist):
        wid = lax.axis_index("s") * 2 + lax.axis_index("c")
        pltpu.sync_copy(yi_hbm.at[pl.ds(wid * SPW * C, SPW * C)], yv)

        iota = lax.iota(jnp.int32, _L)
        zeros = jnp.zeros((_L,), jnp.int32)

        NP = SPW // 2

        def scan_bins(hoff, kk):
            G = zeros
            for l in range(_L):
                G = G + plsc.load_gather(hist, [hoff + iota * _L + l])
            RG = lax.rev(plsc.cumsum(lax.rev(G, (0,))), (0,))
            Dh = plsc.all_reduce_population_count(RG >= kk) - 1
            gt_h = jnp.sum(jnp.where(iota > Dh, G, 0))
            Lv = plsc.load_gather(hist, [hoff + Dh * _L + iota])
            RL = lax.rev(plsc.cumsum(lax.rev(Lv, (0,))), (0,)) + gt_h
            Dl = plsc.all_reduce_population_count(RL >= kk) - 1
            cnt_gt = gt_h + jnp.sum(jnp.where(iota > Dl, Lv, 0))
            return kk - cnt_gt, lax.shift_left(Dh, 4) | Dl

        def per_pair(s, _):
            ya = s * C
            yb = (s + NP) * C

            kka = jnp.full((_L,), K, jnp.int32)
            kkb = kka
            pfa = zeros
            pfb = zeros
            for p in range(4):
                sh = 24 - 8 * p

                def zero_hist(j, _c):
                    hist[pl.ds(j * _L, _L)] = zeros
                    return 0
                lax.fori_loop(0, 2 * _L, zero_hist, 0, unroll=True)

                def count_chunk(c, _c, sh=sh, first=(p == 0), pfa=pfa, pfb=pfb):
                    shv = jnp.full((_L,), sh, jnp.int32)
                    ua = yv[pl.ds(ya + c * _L, _L)]
                    ub = yv[pl.ds(yb + c * _L, _L)]
                    qa = lax.shift_right_logical(ua, shv)
                    qb = lax.shift_right_logical(ub, shv)
                    ba = qa & 255
                    bb = (qb & 255) + 256
                    if first:
                        ca, la = plsc.scan_count(ba)
                        cb, lb = plsc.scan_count(bb)
                    else:
                        e8 = jnp.full((_L,), 8, jnp.int32)
                        ca, la = plsc.scan_count(
                            ba, mask=lax.shift_right_logical(qa, e8) == pfa)
                        cb, lb = plsc.scan_count(
                            bb, mask=lax.shift_right_logical(qb, e8) == pfb)
                    plsc.addupdate_scatter(hist, [ba], ca, mask=la)
                    plsc.addupdate_scatter(hist, [bb], cb, mask=lb)
                    return 0
                lax.fori_loop(0, NCH, count_chunk, 0, unroll=2)

                kka, da = scan_bins(0, kka)
                kkb, db = scan_bins(256, kkb)
                pfa = lax.shift_left(pfa, 8) | da
                pfb = lax.shift_left(pfb, 8) | db

            thra, slotsa = pfa, kka
            thrb, slotsb = pfb, kkb

            def compact(c, carry):
                gba, tia, gbb, tib = carry
                idxv = iota + c * _L
                ua = yv[pl.ds(ya + c * _L, _L)]
                ub = yv[pl.ds(yb + c * _L, _L)]
                gta = ua > thra
                gtb = ub > thrb
                eqa = ua == thra
                eqb = ub == thrb
                gpa = plsc.cumsum(jnp.where(gta, 1, 0))
                gpb = plsc.cumsum(jnp.where(gtb, 1, 0))
                epa = plsc.cumsum(jnp.where(eqa, 1, 0))
                epb = plsc.cumsum(jnp.where(eqb, 1, 0))
                gxa = gpa - jnp.where(gta, 1, 0)
                gxb = gpb - jnp.where(gtb, 1, 0)
                exa = epa - jnp.where(eqa, 1, 0)
                exb = epb - jnp.where(eqb, 1, 0)
                sela = gta | (eqa & (exa + tia < slotsa))
                selb = gtb | (eqb & (exb + tib < slotsb))
                acca = jnp.minimum(exa + tia, slotsa)
                accb = jnp.minimum(exb + tib, slotsb)
                pa = gba + gxa + acca
                pb = gbb + gxb + accb
                plsc.store_scatter(selv, [s * K + pa], idxv, mask=sela)
                plsc.store_scatter(selv, [(s + NP) * K + pb], idxv, mask=selb)
                plsc.store_scatter(excv, [s * K + (c * _L + iota) - pa],
                                   idxv, mask=jnp.logical_not(sela))
                plsc.store_scatter(excv, [(s + NP) * K + (c * _L + iota) - pb],
                                   idxv, mask=jnp.logical_not(selb))
                return (gba + plsc.all_reduce_population_count(gta),
                        tia + plsc.all_reduce_population_count(eqa),
                        gbb + plsc.all_reduce_population_count(gtb),
                        tib + plsc.all_reduce_population_count(eqb))

            lax.fori_loop(0, NCH, compact, (zeros, zeros, zeros, zeros))
            return 0

        lax.fori_loop(0, NP, per_pair, 0)

        pltpu.sync_copy(selv, sel_hbm.at[pl.ds(wid * SPW * K, SPW * K)])
        pltpu.sync_copy(excv, exc_hbm.at[pl.ds(wid * SPW * K, SPW * K)])

    return pl.kernel(
        body,
        out_type=(
            jax.ShapeDtypeStruct((B * K,), jnp.int32),
            jax.ShapeDtypeStruct((B * K,), jnp.int32),
        ),
        mesh=mesh,
        compiler_params=pltpu.CompilerParams(needs_layout_passes=False),
        scratch_types=[
            pltpu.VMEM((SPW * C,), jnp.int32),
            pltpu.VMEM((SPW * K,), jnp.int32),
            pltpu.VMEM((SPW * K,), jnp.int32),
            pltpu.VMEM((2 * _L * _L,), jnp.int32),
        ],
    )


def kernel(x, W1, b1, W2, b2):
    B, C, H, W = x.shape
    K = 384
    xt = x.transpose(2, 3, 0, 1).reshape(H * W, B, C)
    NS = 2
    Bh = B // NS
    sc_call = _make_sc_select(Bh, C, K)
    ys, sels, excs = [], [], []
    for i in range(NS):
        y = _gate_tc(xt, W1, b1, W2, b2, i * Bh, Bh)
        yi = lax.bitcast_convert_type(y, jnp.int32).reshape(Bh * C)
        sel, exc = sc_call(yi)
        ys.append(y)
        sels.append(sel.reshape(Bh, K, 1, 1))
        excs.append(exc.reshape(Bh, K, 1, 1))
    y = jnp.concatenate(ys, axis=0)
    return (
        y.reshape(B, C, 1, 1),
        jnp.concatenate(sels, axis=0),
        jnp.concatenate(excs, axis=0),
    )
